# baseline (speedup 1.0000x reference)
import functools

import jax
import jax.numpy as jnp
from jax import lax
from jax.experimental import pallas as pl
from jax.experimental.pallas import tpu as pltpu
from jax.experimental.pallas import tpu_sc as plsc

B = 16384
D = 64
K = 5
KP = 8
NC = 2
NS = 16
NW = NC * NS
BPW = B // NW
CH = 128
NCH = BPW // CH
NR = 1000000
BLKR = 512
S = 500224
GRID_T = S // BLKR

_mesh = plsc.VectorSubcoreMesh(core_axis_name="c", subcore_axis_name="s",
                               num_cores=NC)


def _pack_body(xa_ref, xb_ref, o_ref):
    o_ref[:, :D] = xa_ref[...]
    o_ref[:, D:] = xb_ref[...]


_tc_pack = pl.pallas_call(
    _pack_body,
    grid=(GRID_T,),
    in_specs=[
        pl.BlockSpec((BLKR, D), lambda i: (i, 0)),
        pl.BlockSpec((BLKR, D), lambda i: (i + GRID_T, 0)),
    ],
    out_specs=pl.BlockSpec((BLKR, 2 * D), lambda i: (i, 0)),
    out_shape=jax.ShapeDtypeStruct((S, 2 * D), jnp.float32),
)


@functools.partial(
    pl.kernel,
    mesh=_mesh,
    compiler_params=pltpu.CompilerParams(use_tc_tiling_on_sc=True),
    out_type=jax.ShapeDtypeStruct((B, 2 * D), jnp.float32),
    scratch_types=[
        pltpu.VMEM((NCH, CH), jnp.int32),
        pltpu.VMEM((CH, 2 * D), jnp.float32),
        pltpu.VMEM((CH, 2 * D), jnp.float32),
        pltpu.VMEM((CH, 2 * D), jnp.float32),
        pltpu.VMEM((CH, 2 * D), jnp.float32),
        pltpu.SemaphoreType.DMA,
        pltpu.SemaphoreType.DMA,
    ],
)
def _sc_gather(idx3, packed, out_hbm, idx_v, b0, b1, b2, b3, gsem, wsem):
    wid = lax.axis_index("s") * NC + lax.axis_index("c")
    base = wid * BPW
    bufs = [b0, b1, b2, b3]

    pltpu.sync_copy(idx3.at[wid], idx_v)
    gs = [pltpu.async_copy(packed.at[idx_v.at[i]], bufs[i], gsem)
          for i in range(NCH)]
    ws = []
    for i in range(NCH):
        gs[i].wait()
        ws.append(pltpu.async_copy(
            bufs[i], out_hbm.at[pl.ds(base + i * CH, CH)], wsem))
    for w in ws:
        w.wait()


def _tc_body(u2_ref, p2_ref, pu_ref, pp_ref, w_ref, b_ref, o_ref):
    u_lo = u2_ref[:, :D]
    u_hi = u2_ref[:, D:]
    p_lo = p2_ref[:, :D]
    p_hi = p2_ref[:, D:]
    u = u_lo + pu_ref[...] * (u_hi - u_lo)
    p = p_lo + pp_ref[...] * (p_hi - p_lo)
    inter = u * p
    o_ref[...] = (
        jnp.dot(inter, w_ref[...], preferred_element_type=jnp.float32)
        + b_ref[...]
    )


_ROWS_BLK = 2048

_tc_logits = pl.pallas_call(
    _tc_body,
    grid=(B // _ROWS_BLK,),
    in_specs=[
        pl.BlockSpec((_ROWS_BLK, 2 * D), lambda i: (i, 0)),
        pl.BlockSpec((_ROWS_BLK, 2 * D), lambda i: (i, 0)),
        pl.BlockSpec((_ROWS_BLK, 1), lambda i: (i, 0)),
        pl.BlockSpec((_ROWS_BLK, 1), lambda i: (i, 0)),
        pl.BlockSpec((D, KP), lambda i: (0, 0)),
        pl.BlockSpec((1, KP), lambda i: (0, 0)),
    ],
    out_specs=pl.BlockSpec((_ROWS_BLK, KP), lambda i: (i, 0)),
    out_shape=jax.ShapeDtypeStruct((B, KP), jnp.float32),
)


def kernel(user, product, user_factors, product_factors, W, b):
    user = user.astype(jnp.int32)
    product = product.astype(jnp.int32)
    su = (user >= S).astype(jnp.int32)
    sp = (product >= S).astype(jnp.int32)
    u3 = (user - S * su).reshape(NW, NCH, CH)
    p3 = (product - S * sp).reshape(NW, NCH, CH)

    u_packed = _tc_pack(user_factors, user_factors)
    u2g = _sc_gather(u3, u_packed)
    p_packed = _tc_pack(product_factors, product_factors)
    p2g = _sc_gather(p3, p_packed)

    pu = su.astype(jnp.float32).reshape(B, 1)
    pp = sp.astype(jnp.float32).reshape(B, 1)
    wt = jnp.zeros((D, KP), jnp.float32).at[:, :K].set(W.T)
    bp = jnp.zeros((1, KP), jnp.float32).at[0, :K].set(b)
    out = _tc_logits(u2g, p2g, pu, pp, wt, bp)
    return out[:, :K]


# speedup vs baseline: 2.7193x; 2.7193x over previous
import functools

import jax
import jax.numpy as jnp
from jax import lax
from jax.experimental import pallas as pl
from jax.experimental.pallas import tpu as pltpu
from jax.experimental.pallas import tpu_sc as plsc

B = 16384
D = 64
K = 5
KP = 8
NC = 2
NS = 16
NW = NC * NS
BPW = B // NW
CH = 64
NCH = BPW // CH
GL = 8

_mesh = plsc.VectorSubcoreMesh(core_axis_name="c", subcore_axis_name="s",
                               num_cores=NC)


@functools.partial(
    pl.kernel,
    mesh=_mesh,
    compiler_params=pltpu.CompilerParams(use_tc_tiling_on_sc=True),
    out_type=jax.ShapeDtypeStruct((B, D), jnp.float32),
    scratch_types=[
        pltpu.VMEM((NCH, CH), jnp.int32),
        pltpu.VMEM((NCH, CH), jnp.int32),
        pltpu.VMEM((CH, GL, D), jnp.float32),
        pltpu.VMEM((CH, D), jnp.float32),
        pltpu.SemaphoreType.DMA,
        pltpu.SemaphoreType.DMA,
    ],
)
def _sc_gather_rows(g3, s3, tab, out_hbm, g_v, s_v, grp_v, rows_v,
                    gsem, wsem):
    wid = lax.axis_index("s") * NC + lax.axis_index("c")
    base = wid * BPW

    pltpu.sync_copy(g3.at[wid], g_v)
    pltpu.sync_copy(s3.at[wid], s_v)

    for c in range(NCH):
        def fire_body(g, carry, c=c):
            gv = g_v[c, pl.ds(g * 16, 16)]
            for lane in range(16):
                start = pl.multiple_of(gv[lane] * GL, GL)
                pltpu.async_copy(
                    tab.at[pl.ds(start, GL)], grp_v.at[g * 16 + lane], gsem)
            return carry

        lax.fori_loop(0, CH // 16, fire_body, 0)

        for j in range(CH):
            pltpu.make_async_copy(
                tab.at[pl.ds(0, GL)], grp_v.at[0], gsem).wait()

        def extract_body(g, carry, c=c):
            sv = s_v[c, pl.ds(g * 16, 16)]
            for lane in range(16):
                j = g * 16 + lane
                sub = sv[lane]
                for q in range(D // 16):
                    sl = pl.ds(q * 16, 16)
                    rows_v[j, sl] = grp_v[j, sub, sl]
            return carry

        lax.fori_loop(0, CH // 16, extract_body, 0)
        pltpu.async_copy(
            rows_v, out_hbm.at[pl.ds(base + c * CH, CH)], wsem).wait()


def _tc_body(u_ref, p_ref, w_ref, b_ref, o_ref):
    inter = u_ref[...] * p_ref[...]
    o_ref[...] = (
        jnp.dot(inter, w_ref[...], preferred_element_type=jnp.float32)
        + b_ref[...]
    )


_ROWS_BLK = 2048

_tc_logits = pl.pallas_call(
    _tc_body,
    grid=(B // _ROWS_BLK,),
    in_specs=[
        pl.BlockSpec((_ROWS_BLK, D), lambda i: (i, 0)),
        pl.BlockSpec((_ROWS_BLK, D), lambda i: (i, 0)),
        pl.BlockSpec((D, KP), lambda i: (0, 0)),
        pl.BlockSpec((1, KP), lambda i: (0, 0)),
    ],
    out_specs=pl.BlockSpec((_ROWS_BLK, KP), lambda i: (i, 0)),
    out_shape=jax.ShapeDtypeStruct((B, KP), jnp.float32),
)


def kernel(user, product, user_factors, product_factors, W, b):
    user = user.astype(jnp.int32)
    product = product.astype(jnp.int32)
    ug3 = (user >> 3).reshape(NW, NCH, CH)
    us3 = (user & 7).reshape(NW, NCH, CH)
    pg3 = (product >> 3).reshape(NW, NCH, CH)
    ps3 = (product & 7).reshape(NW, NCH, CH)

    u_rows = _sc_gather_rows(ug3, us3, user_factors)
    p_rows = _sc_gather_rows(pg3, ps3, product_factors)

    wt = jnp.zeros((D, KP), jnp.float32).at[:, :K].set(W.T)
    bp = jnp.zeros((1, KP), jnp.float32).at[0, :K].set(b)
    out = _tc_logits(u_rows, p_rows, wt, bp)
    return out[:, :K]


# speedup vs baseline: 3.6993x; 1.3604x over previous
import functools

import jax
import jax.numpy as jnp
from jax import lax
from jax.experimental import pallas as pl
from jax.experimental.pallas import tpu as pltpu
from jax.experimental.pallas import tpu_sc as plsc

B = 16384
D = 64
K = 5
KP = 8
NC = 2
NS = 16
NW = NC * NS
BPW = B // NW
CH = 64
NCH = BPW // CH
GL = 8

_mesh = plsc.VectorSubcoreMesh(core_axis_name="c", subcore_axis_name="s",
                               num_cores=NC)


@functools.partial(
    pl.kernel,
    mesh=_mesh,
    compiler_params=pltpu.CompilerParams(use_tc_tiling_on_sc=True),
    out_type=jax.ShapeDtypeStruct((B, D), jnp.float32),
    scratch_types=[
        pltpu.VMEM((NCH, CH), jnp.int32),
        pltpu.VMEM((NCH, CH), jnp.int32),
        pltpu.VMEM((CH, GL, D), jnp.float32),
        pltpu.VMEM((CH, D), jnp.float32),
        pltpu.SemaphoreType.DMA,
        pltpu.SemaphoreType.DMA,
    ],
)
def _sc_gather_rows(g3, s3, tab, out_hbm, g_v, s_v, grp_v, rows_v,
                    gsem, wsem):
    wid = lax.axis_index("s") * NC + lax.axis_index("c")
    base = wid * BPW

    pltpu.sync_copy(g3.at[wid], g_v)
    pltpu.sync_copy(s3.at[wid], s_v)

    for c in range(NCH):
        def fire_body(g, carry, c=c):
            gv = g_v[c, pl.ds(g * 16, 16)]
            for lane in range(16):
                pltpu.async_copy(
                    tab.at[gv[lane]], grp_v.at[g * 16 + lane], gsem)
            return carry

        lax.fori_loop(0, CH // 16, fire_body, 0)

        for j in range(CH):
            pltpu.make_async_copy(
                tab.at[0], grp_v.at[0], gsem).wait()

        def extract_body(g, carry, c=c):
            sv = s_v[c, pl.ds(g * 16, 16)]
            for lane in range(16):
                j = g * 16 + lane
                sub = sv[lane]
                for q in range(D // 16):
                    sl = pl.ds(q * 16, 16)
                    rows_v[j, sl] = grp_v[j, sub, sl]
            return carry

        lax.fori_loop(0, CH // 16, extract_body, 0)
        pltpu.async_copy(
            rows_v, out_hbm.at[pl.ds(base + c * CH, CH)], wsem).wait()


def _tc_body(u_ref, p_ref, w_ref, b_ref, o_ref):
    inter = u_ref[...] * p_ref[...]
    o_ref[...] = (
        jnp.dot(inter, w_ref[...], preferred_element_type=jnp.float32)
        + b_ref[...]
    )


_ROWS_BLK = 2048

_tc_logits = pl.pallas_call(
    _tc_body,
    grid=(B // _ROWS_BLK,),
    in_specs=[
        pl.BlockSpec((_ROWS_BLK, D), lambda i: (i, 0)),
        pl.BlockSpec((_ROWS_BLK, D), lambda i: (i, 0)),
        pl.BlockSpec((D, KP), lambda i: (0, 0)),
        pl.BlockSpec((1, KP), lambda i: (0, 0)),
    ],
    out_specs=pl.BlockSpec((_ROWS_BLK, KP), lambda i: (i, 0)),
    out_shape=jax.ShapeDtypeStruct((B, KP), jnp.float32),
)


def kernel(user, product, user_factors, product_factors, W, b):
    user = user.astype(jnp.int32)
    product = product.astype(jnp.int32)
    ug3 = (user >> 3).reshape(NW, NCH, CH)
    us3 = (user & 7).reshape(NW, NCH, CH)
    pg3 = (product >> 3).reshape(NW, NCH, CH)
    ps3 = (product & 7).reshape(NW, NCH, CH)

    uf8 = user_factors.reshape(125000, GL, D)
    pf8 = product_factors.reshape(125000, GL, D)
    u_rows = _sc_gather_rows(ug3, us3, uf8)
    p_rows = _sc_gather_rows(pg3, ps3, pf8)

    wt = jnp.zeros((D, KP), jnp.float32).at[:, :K].set(W.T)
    bp = jnp.zeros((1, KP), jnp.float32).at[0, :K].set(b)
    out = _tc_logits(u_rows, p_rows, wt, bp)
    return out[:, :K]
